# baseline (speedup 1.0000x reference)
import functools

import jax
import jax.numpy as jnp
from jax.experimental import pallas as pl
from jax.experimental.pallas import tpu as pltpu

N = 4096
D = 128
C = 100
K = 100
KC = 20
NUM_TRUE_TH = 90
SCORE_TH = 0.99

ROWS = 256

_MASK31 = 0x7FFFFFFF
_HOLE = -2**31


def _py_key(x):
    import struct
    i = struct.unpack('<i', struct.pack('<f', x))[0]
    return i ^ 0x7FFFFFFF if i < 0 else i


_LO_INIT = _py_key(-1.5)
_HI_INIT = _py_key(1.5)


def _to_key(v):
    i = jax.lax.bitcast_convert_type(v, jnp.int32)
    return jnp.where(i < 0, i ^ _MASK31, i)


def _from_key(k):
    i = jnp.where(k < 0, k ^ _MASK31, k)
    return jax.lax.bitcast_convert_type(i, jnp.float32)


def _shift_left(x, d, fill, iota):
    n = x.shape[-1]
    return jnp.where(iota < n - d, pltpu.roll(x, n - d, 1), fill)


def _shift_right(x, d, fill, iota):
    return jnp.where(iota >= d, pltpu.roll(x, d, 1), fill)


def _topk_body(feas_blk, feas_all, scoresT, scores_blk,
               sk_ref, nt_ref, it_ref, kbuf, zbuf):
    R = ROWS
    sim = jax.lax.dot_general(
        feas_blk[...], feas_all[...],
        (((1,), (1,)), ((), ())),
        preferred_element_type=jnp.float32,
        precision=jax.lax.Precision.DEFAULT,
    )
    kbuf[...] = _to_key(sim)

    iota_l = jax.lax.broadcasted_iota(jnp.int32, (R, N), 1)

    keys0 = kbuf[...]
    ones = jnp.int32(1)
    c0 = jnp.sum((keys0 >= 0).astype(jnp.int32), axis=1, keepdims=True)
    pred0 = c0 >= K
    lo = jnp.where(pred0, jnp.int32(0), jnp.int32(_LO_INIT))
    hi = jnp.where(pred0, jnp.int32(_HI_INIT), jnp.int32(0))
    cnt_hi = jnp.where(pred0, jnp.int32(0), c0)

    def search(i, carry):
        lo, hi, cnt_hi = carry
        mid = lo + jnp.maximum((hi - lo) >> 1, ones)
        keys = kbuf[...]
        c = jnp.sum((keys >= mid).astype(jnp.int32), axis=1, keepdims=True)
        pred = c >= K
        return (jnp.where(pred, mid, lo),
                jnp.where(pred, hi, mid),
                jnp.where(pred, cnt_hi, c))

    lo, hi, cnt_hi = jax.lax.fori_loop(0, 31, search, (lo, hi, cnt_hi),
                                       unroll=False)
    tau = lo
    need = K - cnt_hi

    keys = kbuf[...]
    gt = keys > tau
    eq = keys == tau
    combo = (gt.astype(jnp.int32) << 13) | eq.astype(jnp.int32)
    zbuf[...] = combo

    def scan_step(i, carry):
        d = jax.lax.shift_left(ones, i)
        c = zbuf[...]
        zbuf[...] = c + _shift_right(c, d, jnp.int32(0), iota_l)
        return carry

    jax.lax.fori_loop(0, 12, scan_step, 0, unroll=False)
    excl = zbuf[...] - combo
    gtb = excl >> 13
    eqb = excl & jnp.int32(8191)

    incl = gt | (eq & (eqb < need))

    st = scoresT[...]
    mt = jnp.max(st, axis=0, keepdims=True)
    iota_c = jax.lax.broadcasted_iota(jnp.int32, (C, N), 0)
    labels_all = jnp.min(jnp.where(st == mt, iota_c, C), axis=0,
                         keepdims=True)
    mk = jnp.max(keys, axis=1, keepdims=True)
    j0 = jnp.min(jnp.where(keys == mk, iota_l, N), axis=1, keepdims=True)
    top1_label = jnp.sum(jnp.where(iota_l == j0, labels_all, 0), axis=1,
                         keepdims=True)
    nt = jnp.sum((incl & (labels_all == top1_label)).astype(jnp.int32),
                 axis=1, keepdims=True)
    conf = jnp.max(scores_blk[...], axis=1, keepdims=True)
    nt_ref[...] = nt
    it_ref[...] = (nt >= NUM_TRUE_TH) & (conf > SCORE_TH)

    dest = gtb + jnp.minimum(eqb, need)
    kbuf[...] = jnp.where(incl, keys, _HOLE)
    zbuf[...] = jnp.where(incl, iota_l - dest, jnp.int32(-1))

    def route_step(i, carry):
        d = jax.lax.shift_left(ones, i)
        k = kbuf[...]
        z = zbuf[...]
        pk = _shift_left(k, d, _HOLE, iota_l)
        pz = _shift_left(z, d, jnp.int32(-1), iota_l)
        inc = (pz >= 0) & ((pz & d) != 0)
        leave = (z >= 0) & ((z & d) != 0)
        kbuf[...] = jnp.where(inc, pk, jnp.where(leave, _HOLE, k))
        zbuf[...] = jnp.where(inc, pz - d, jnp.where(leave, jnp.int32(-1), z))
        return carry

    jax.lax.fori_loop(0, 12, route_step, 0, unroll=False)

    x = kbuf[:, :128]
    lane = jax.lax.broadcasted_iota(jnp.int32, (R, 128), 1)
    for k_ in range(1, 8):
        for j_ in range(k_ - 1, -1, -1):
            d = 1 << j_
            p = jnp.where((lane & d) == 0,
                          pltpu.roll(x, 128 - d, 1), pltpu.roll(x, d, 1))
            if k_ < 7:
                up = ((lane >> k_) & 1) == 0
            else:
                up = jnp.full((R, 128), True)
            first = (lane & d) == 0
            x = jnp.where(up == first, jnp.maximum(x, p), jnp.minimum(x, p))

    sk_ref[...] = _from_key(x[:, :K])


def _centers_body(scoresT_ref, feas_ref, out_ref, sbuf, wbuf):
    sbuf[...] = scoresT_ref[...]
    wbuf[...] = jnp.zeros((C, N), jnp.float32)
    iota_l = jax.lax.broadcasted_iota(jnp.int32, (C, N), 1)

    def body(r, carry):
        s = sbuf[...]
        m = jnp.max(s, axis=1, keepdims=True)
        j = jnp.min(jnp.where(s == m, iota_l, N), axis=1, keepdims=True)
        onehot = iota_l == j
        sbuf[...] = jnp.where(onehot, -1.0, s)
        wbuf[...] += onehot.astype(jnp.float32)
        return carry

    jax.lax.fori_loop(0, KC, body, 0, unroll=False)

    acc = jax.lax.dot_general(
        wbuf[...], feas_ref[...],
        (((1,), (0,)), ((), ())),
        preferred_element_type=jnp.float32,
        precision=jax.lax.Precision.HIGHEST,
    )
    out_ref[...] = acc / jnp.float32(KC)


@functools.partial(jax.jit, static_argnames=("interpret",))
def kernel(feas_sim, scores, interpret=False):
    scoresT = scores.T

    grid = N // ROWS
    scores_k, num_true, idx_true = pl.pallas_call(
        _topk_body,
        grid=(grid,),
        in_specs=[
            pl.BlockSpec((ROWS, D), lambda i: (i, 0)),
            pl.BlockSpec((N, D), lambda i: (0, 0)),
            pl.BlockSpec((C, N), lambda i: (0, 0)),
            pl.BlockSpec((ROWS, C), lambda i: (i, 0)),
        ],
        out_specs=[
            pl.BlockSpec((ROWS, K), lambda i: (i, 0)),
            pl.BlockSpec((ROWS, 1), lambda i: (i, 0)),
            pl.BlockSpec((ROWS, 1), lambda i: (i, 0)),
        ],
        out_shape=[
            jax.ShapeDtypeStruct((N, K), jnp.float32),
            jax.ShapeDtypeStruct((N, 1), jnp.int32),
            jax.ShapeDtypeStruct((N, 1), jnp.bool_),
        ],
        scratch_shapes=[
            pltpu.VMEM((ROWS, N), jnp.int32),
            pltpu.VMEM((ROWS, N), jnp.int32),
        ],
        interpret=interpret,
    )(feas_sim, feas_sim, scoresT, scores)

    centers = pl.pallas_call(
        _centers_body,
        out_shape=jax.ShapeDtypeStruct((C, D), jnp.float32),
        scratch_shapes=[
            pltpu.VMEM((C, N), jnp.float32),
            pltpu.VMEM((C, N), jnp.float32),
        ],
        interpret=interpret,
    )(scoresT, feas_sim)

    return centers, scores_k, num_true[:, 0], idx_true[:, 0]


# speedup vs baseline: 2.2336x; 2.2336x over previous
import functools

import jax
import jax.numpy as jnp
from jax.experimental import pallas as pl
from jax.experimental.pallas import tpu as pltpu

N = 4096
D = 128
C = 100
K = 100
KC = 20
NUM_TRUE_TH = 90
SCORE_TH = 0.99

ROWS = 256

_MASK31 = 0x7FFFFFFF
_HOLE = -2**31


def _py_key(x):
    import struct
    i = struct.unpack('<i', struct.pack('<f', x))[0]
    return i ^ 0x7FFFFFFF if i < 0 else i


_LO_INIT = _py_key(-1.5)
_HI_INIT = _py_key(1.5)


def _to_key(v):
    i = jax.lax.bitcast_convert_type(v, jnp.int32)
    return jnp.where(i < 0, i ^ _MASK31, i)


def _from_key(k):
    i = jnp.where(k < 0, k ^ _MASK31, k)
    return jax.lax.bitcast_convert_type(i, jnp.float32)


def _shift_left(x, d, fill, iota):
    n = x.shape[-1]
    return jnp.where(iota < n - d, pltpu.roll(x, n - d, 1), fill)


def _shift_right(x, d, fill, iota):
    return jnp.where(iota >= d, pltpu.roll(x, d, 1), fill)


def _topk_body(feas_blk, feas_all, scoresT, scores_blk,
               sk_ref, nt_ref, it_ref, kbuf, zbuf):
    R = ROWS
    sim = jax.lax.dot_general(
        feas_blk[...], feas_all[...],
        (((1,), (1,)), ((), ())),
        preferred_element_type=jnp.float32,
        precision=jax.lax.Precision.DEFAULT,
    )
    kbuf[...] = _to_key(sim)

    iota_l = jax.lax.broadcasted_iota(jnp.int32, (R, N), 1)

    keys0 = kbuf[...]
    ones = jnp.int32(1)
    c0 = jnp.sum((keys0 >= 0).astype(jnp.int32), axis=1, keepdims=True)
    pred0 = c0 >= K
    lo = jnp.where(pred0, jnp.int32(0), jnp.int32(_LO_INIT))
    hi = jnp.where(pred0, jnp.int32(_HI_INIT), jnp.int32(0))
    cnt_hi = jnp.where(pred0, jnp.int32(0), c0)

    def search(i, carry):
        lo, hi, cnt_hi = carry
        mid = lo + jnp.maximum((hi - lo) >> 1, ones)
        keys = kbuf[...]
        c = jnp.sum((keys >= mid).astype(jnp.int32), axis=1, keepdims=True)
        pred = c >= K
        return (jnp.where(pred, mid, lo),
                jnp.where(pred, hi, mid),
                jnp.where(pred, cnt_hi, c))

    lo, hi, cnt_hi = jax.lax.fori_loop(0, 31, search, (lo, hi, cnt_hi),
                                       unroll=False)
    tau = lo
    need = K - cnt_hi

    keys = kbuf[...]
    gt = keys > tau
    eq = keys == tau
    combo = (gt.astype(jnp.int32) << 13) | eq.astype(jnp.int32)
    zbuf[...] = combo

    c = zbuf[...]
    for b_ in range(12):
        d_ = 1 << b_
        c = c + _shift_right(c, d_, jnp.int32(0), iota_l)
    zbuf[...] = c
    excl = zbuf[...] - combo
    gtb = excl >> 13
    eqb = excl & jnp.int32(8191)

    incl = gt | (eq & (eqb < need))

    st = scoresT[...]
    mt = jnp.max(st, axis=0, keepdims=True)
    iota_c = jax.lax.broadcasted_iota(jnp.int32, (C, N), 0)
    labels_all = jnp.min(jnp.where(st == mt, iota_c, C), axis=0,
                         keepdims=True)
    mk = jnp.max(keys, axis=1, keepdims=True)
    j0 = jnp.min(jnp.where(keys == mk, iota_l, N), axis=1, keepdims=True)
    top1_label = jnp.sum(jnp.where(iota_l == j0, labels_all, 0), axis=1,
                         keepdims=True)
    nt = jnp.sum((incl & (labels_all == top1_label)).astype(jnp.int32),
                 axis=1, keepdims=True)
    conf = jnp.max(scores_blk[...], axis=1, keepdims=True)
    nt_ref[...] = nt
    it_ref[...] = (nt >= NUM_TRUE_TH) & (conf > SCORE_TH)

    dest = gtb + jnp.minimum(eqb, need)
    kbuf[...] = jnp.where(incl, keys, _HOLE)
    zbuf[...] = jnp.where(incl, iota_l - dest, jnp.int32(-1))

    k = kbuf[...]
    z = zbuf[...]
    for b_ in range(12):
        d_ = 1 << b_
        pk = _shift_left(k, d_, _HOLE, iota_l)
        pz = _shift_left(z, d_, jnp.int32(-1), iota_l)
        inc = (pz >= 0) & ((pz & d_) != 0)
        leave = (z >= 0) & ((z & d_) != 0)
        k = jnp.where(inc, pk, jnp.where(leave, _HOLE, k))
        z = jnp.where(inc, pz - d_, jnp.where(leave, jnp.int32(-1), z))

    x = k[:, :128]
    lane = jax.lax.broadcasted_iota(jnp.int32, (R, 128), 1)
    for k_ in range(1, 8):
        for j_ in range(k_ - 1, -1, -1):
            d = 1 << j_
            p = jnp.where((lane & d) == 0,
                          pltpu.roll(x, 128 - d, 1), pltpu.roll(x, d, 1))
            if k_ < 7:
                up = ((lane >> k_) & 1) == 0
            else:
                up = jnp.full((R, 128), True)
            first = (lane & d) == 0
            x = jnp.where(up == first, jnp.maximum(x, p), jnp.minimum(x, p))

    sk_ref[...] = _from_key(x[:, :K])


def _centers_body(scoresT_ref, feas_ref, out_ref, sbuf, wbuf):
    sbuf[...] = scoresT_ref[...]
    wbuf[...] = jnp.zeros((C, N), jnp.float32)
    iota_l = jax.lax.broadcasted_iota(jnp.int32, (C, N), 1)

    def body(r, carry):
        s = sbuf[...]
        m = jnp.max(s, axis=1, keepdims=True)
        j = jnp.min(jnp.where(s == m, iota_l, N), axis=1, keepdims=True)
        onehot = iota_l == j
        sbuf[...] = jnp.where(onehot, -1.0, s)
        wbuf[...] += onehot.astype(jnp.float32)
        return carry

    jax.lax.fori_loop(0, KC, body, 0, unroll=False)

    acc = jax.lax.dot_general(
        wbuf[...], feas_ref[...],
        (((1,), (0,)), ((), ())),
        preferred_element_type=jnp.float32,
        precision=jax.lax.Precision.HIGHEST,
    )
    out_ref[...] = acc / jnp.float32(KC)


@functools.partial(jax.jit, static_argnames=("interpret",))
def kernel(feas_sim, scores, interpret=False):
    scoresT = scores.T

    grid = N // ROWS
    scores_k, num_true, idx_true = pl.pallas_call(
        _topk_body,
        grid=(grid,),
        in_specs=[
            pl.BlockSpec((ROWS, D), lambda i: (i, 0)),
            pl.BlockSpec((N, D), lambda i: (0, 0)),
            pl.BlockSpec((C, N), lambda i: (0, 0)),
            pl.BlockSpec((ROWS, C), lambda i: (i, 0)),
        ],
        out_specs=[
            pl.BlockSpec((ROWS, K), lambda i: (i, 0)),
            pl.BlockSpec((ROWS, 1), lambda i: (i, 0)),
            pl.BlockSpec((ROWS, 1), lambda i: (i, 0)),
        ],
        out_shape=[
            jax.ShapeDtypeStruct((N, K), jnp.float32),
            jax.ShapeDtypeStruct((N, 1), jnp.int32),
            jax.ShapeDtypeStruct((N, 1), jnp.bool_),
        ],
        scratch_shapes=[
            pltpu.VMEM((ROWS, N), jnp.int32),
            pltpu.VMEM((ROWS, N), jnp.int32),
        ],
        interpret=interpret,
    )(feas_sim, feas_sim, scoresT, scores)

    centers = pl.pallas_call(
        _centers_body,
        out_shape=jax.ShapeDtypeStruct((C, D), jnp.float32),
        scratch_shapes=[
            pltpu.VMEM((C, N), jnp.float32),
            pltpu.VMEM((C, N), jnp.float32),
        ],
        interpret=interpret,
    )(scoresT, feas_sim)

    return centers, scores_k, num_true[:, 0], idx_true[:, 0]


# speedup vs baseline: 2.4059x; 1.0771x over previous
import functools

import jax
import jax.numpy as jnp
from jax.experimental import pallas as pl
from jax.experimental.pallas import tpu as pltpu

N = 4096
D = 128
C = 100
K = 100
KC = 20
NUM_TRUE_TH = 90
SCORE_TH = 0.99

ROWS = 256

_MASK31 = 0x7FFFFFFF
_HOLE = -2**31


def _py_key(x):
    import struct
    i = struct.unpack('<i', struct.pack('<f', x))[0]
    return i ^ 0x7FFFFFFF if i < 0 else i


_LO_INIT = _py_key(-1.5)
_HI_INIT = _py_key(1.5)


def _to_key(v):
    i = jax.lax.bitcast_convert_type(v, jnp.int32)
    return jnp.where(i < 0, i ^ _MASK31, i)


def _from_key(k):
    i = jnp.where(k < 0, k ^ _MASK31, k)
    return jax.lax.bitcast_convert_type(i, jnp.float32)


def _shift_left(x, d, fill, iota):
    n = x.shape[-1]
    return jnp.where(iota < n - d, pltpu.roll(x, n - d, 1), fill)


def _shift_right(x, d, fill, iota):
    return jnp.where(iota >= d, pltpu.roll(x, d, 1), fill)


def _topk_body(feas_blk, feas_all, scoresT, scores_blk,
               sk_ref, nt_ref, it_ref, kbuf, zbuf):
    R = ROWS
    sim = jax.lax.dot_general(
        feas_blk[...], feas_all[...],
        (((1,), (1,)), ((), ())),
        preferred_element_type=jnp.float32,
        precision=jax.lax.Precision.DEFAULT,
    )
    kbuf[...] = _to_key(sim)

    iota_l = jax.lax.broadcasted_iota(jnp.int32, (R, N), 1)

    keys0 = kbuf[...]
    ones = jnp.int32(1)
    c0 = jnp.sum((keys0 >= 0).astype(jnp.int32), axis=1, keepdims=True)
    pred0 = c0 >= K
    lo = jnp.where(pred0, jnp.int32(0), jnp.int32(_LO_INIT))
    hi = jnp.where(pred0, jnp.int32(_HI_INIT), jnp.int32(0))
    cnt_hi = jnp.where(pred0, jnp.int32(0), c0)

    def search(i, carry):
        lo, hi, cnt_hi = carry
        mid = lo + jnp.maximum((hi - lo) >> 1, ones)
        keys = kbuf[...]
        c = jnp.sum((keys >= mid).astype(jnp.int32), axis=1, keepdims=True)
        pred = c >= K
        return (jnp.where(pred, mid, lo),
                jnp.where(pred, hi, mid),
                jnp.where(pred, cnt_hi, c))

    lo, hi, cnt_hi = jax.lax.fori_loop(0, 31, search, (lo, hi, cnt_hi),
                                       unroll=False)
    tau = lo
    need = K - cnt_hi

    keys = kbuf[...]
    gt = keys > tau
    eq = keys == tau
    combo = (gt.astype(jnp.int32) << 13) | eq.astype(jnp.int32)
    zbuf[...] = combo

    c = zbuf[...]
    for b_ in range(12):
        d_ = 1 << b_
        c = c + _shift_right(c, d_, jnp.int32(0), iota_l)
    zbuf[...] = c
    excl = zbuf[...] - combo
    gtb = excl >> 13
    eqb = excl & jnp.int32(8191)

    incl = gt | (eq & (eqb < need))

    st = scoresT[...]
    mt = jnp.max(st, axis=0, keepdims=True)
    iota_c = jax.lax.broadcasted_iota(jnp.int32, (C, N), 0)
    labels_all = jnp.min(jnp.where(st == mt, iota_c, C), axis=0,
                         keepdims=True)
    mk = jnp.max(keys, axis=1, keepdims=True)
    j0 = jnp.min(jnp.where(keys == mk, iota_l, N), axis=1, keepdims=True)
    top1_label = jnp.sum(jnp.where(iota_l == j0, labels_all, 0), axis=1,
                         keepdims=True)
    nt = jnp.sum((incl & (labels_all == top1_label)).astype(jnp.int32),
                 axis=1, keepdims=True)
    conf = jnp.max(scores_blk[...], axis=1, keepdims=True)
    nt_ref[...] = nt
    it_ref[...] = (nt >= NUM_TRUE_TH) & (conf > SCORE_TH)

    dest = gtb + jnp.minimum(eqb, need)
    k = keys
    z = jnp.where(incl, iota_l - dest, jnp.int32(0))
    for b_ in range(12):
        d_ = 1 << b_
        pk = _shift_left(k, d_, _HOLE, iota_l)
        pz = _shift_left(z, d_, jnp.int32(0), iota_l)
        inc = (pz & d_) != 0
        leave = (z & d_) != 0
        k = jnp.where(inc, pk, k)
        z = jnp.where(inc, pz - d_, jnp.where(leave, jnp.int32(0), z))

    lane = jax.lax.broadcasted_iota(jnp.int32, (R, 128), 1)
    x = jnp.where(lane < K, k[:, :128], _HOLE)
    for k_ in range(1, 8):
        for j_ in range(k_ - 1, -1, -1):
            d = 1 << j_
            p = jnp.where((lane & d) == 0,
                          pltpu.roll(x, 128 - d, 1), pltpu.roll(x, d, 1))
            if k_ < 7:
                up = ((lane >> k_) & 1) == 0
            else:
                up = jnp.full((R, 128), True)
            first = (lane & d) == 0
            x = jnp.where(up == first, jnp.maximum(x, p), jnp.minimum(x, p))

    sk_ref[...] = _from_key(x[:, :K])


def _centers_body(scoresT_ref, feas_ref, out_ref, sbuf, wbuf):
    sbuf[...] = scoresT_ref[...]
    wbuf[...] = jnp.zeros((C, N), jnp.float32)
    iota_l = jax.lax.broadcasted_iota(jnp.int32, (C, N), 1)

    def body(r, carry):
        s = sbuf[...]
        m = jnp.max(s, axis=1, keepdims=True)
        j = jnp.min(jnp.where(s == m, iota_l, N), axis=1, keepdims=True)
        onehot = iota_l == j
        sbuf[...] = jnp.where(onehot, -1.0, s)
        wbuf[...] += onehot.astype(jnp.float32)
        return carry

    jax.lax.fori_loop(0, KC, body, 0, unroll=False)

    acc = jax.lax.dot_general(
        wbuf[...], feas_ref[...],
        (((1,), (0,)), ((), ())),
        preferred_element_type=jnp.float32,
        precision=jax.lax.Precision.HIGHEST,
    )
    out_ref[...] = acc / jnp.float32(KC)


@functools.partial(jax.jit, static_argnames=("interpret",))
def kernel(feas_sim, scores, interpret=False):
    scoresT = scores.T

    grid = N // ROWS
    scores_k, num_true, idx_true = pl.pallas_call(
        _topk_body,
        grid=(grid,),
        in_specs=[
            pl.BlockSpec((ROWS, D), lambda i: (i, 0)),
            pl.BlockSpec((N, D), lambda i: (0, 0)),
            pl.BlockSpec((C, N), lambda i: (0, 0)),
            pl.BlockSpec((ROWS, C), lambda i: (i, 0)),
        ],
        out_specs=[
            pl.BlockSpec((ROWS, K), lambda i: (i, 0)),
            pl.BlockSpec((ROWS, 1), lambda i: (i, 0)),
            pl.BlockSpec((ROWS, 1), lambda i: (i, 0)),
        ],
        out_shape=[
            jax.ShapeDtypeStruct((N, K), jnp.float32),
            jax.ShapeDtypeStruct((N, 1), jnp.int32),
            jax.ShapeDtypeStruct((N, 1), jnp.bool_),
        ],
        scratch_shapes=[
            pltpu.VMEM((ROWS, N), jnp.int32),
            pltpu.VMEM((ROWS, N), jnp.int32),
        ],
        interpret=interpret,
    )(feas_sim, feas_sim, scoresT, scores)

    centers = pl.pallas_call(
        _centers_body,
        out_shape=jax.ShapeDtypeStruct((C, D), jnp.float32),
        scratch_shapes=[
            pltpu.VMEM((C, N), jnp.float32),
            pltpu.VMEM((C, N), jnp.float32),
        ],
        interpret=interpret,
    )(scoresT, feas_sim)

    return centers, scores_k, num_true[:, 0], idx_true[:, 0]
